# Initial kernel scaffold; baseline (speedup 1.0000x reference)
import functools

import jax
import jax.numpy as jnp
from jax import lax
from jax.experimental import pallas as pl
from jax.experimental.pallas import tpu as pltpu
from jax.experimental.pallas import tpu_sc as plsc

N = 10000
E = 320000
D = 128
DE = 16

_NSUB = 16
_EPS_PER = E // _NSUB
_K = 80
_NCH = _EPS_PER // _K
_ZROWS = N // _NSUB
_ZB = 125


def _eproj(ea_all, w_edge, b_edge):
    bm = 1280
    grid = (ea_all.shape[0] // bm,)

    def body(ea_ref, w_ref, b_ref, o_ref):
        o_ref[...] = (
            jnp.dot(ea_ref[...], w_ref[...], preferred_element_type=jnp.float32)
            + b_ref[...]
        )

    return pl.pallas_call(
        body,
        grid=grid,
        in_specs=[
            pl.BlockSpec((bm, DE), lambda i: (i, 0)),
            pl.BlockSpec((DE, D), lambda i: (0, 0)),
            pl.BlockSpec((1, D), lambda i: (0, 0)),
        ],
        out_specs=pl.BlockSpec((bm, D), lambda i: (i, 0)),
        out_shape=jax.ShapeDtypeStruct((ea_all.shape[0], D), jnp.float32),
    )(ea_all, w_edge, b_edge.reshape(1, D))


def _out_transform(x_stack, agg_stack, w_nn, b_nn, eps, relu):
    bm = 1000
    grid = (x_stack.shape[0] // bm,)

    def body(x_ref, a_ref, w_ref, b_ref, e_ref, o_ref):
        z = (1.0 + e_ref[0, 0]) * x_ref[...] + a_ref[...]
        r = jnp.dot(z, w_ref[...], preferred_element_type=jnp.float32) + b_ref[...]
        o_ref[...] = jnp.maximum(r, 0.0) if relu else r

    return pl.pallas_call(
        body,
        grid=grid,
        in_specs=[
            pl.BlockSpec((bm, D), lambda i: (i, 0)),
            pl.BlockSpec((bm, D), lambda i: (i, 0)),
            pl.BlockSpec((D, D), lambda i: (0, 0)),
            pl.BlockSpec((1, D), lambda i: (0, 0)),
            pl.BlockSpec((1, 1), lambda i: (0, 0)),
        ],
        out_specs=pl.BlockSpec((bm, D), lambda i: (i, 0)),
        out_shape=jax.ShapeDtypeStruct((x_stack.shape[0], D), jnp.float32),
    )(x_stack, agg_stack, w_nn, b_nn.reshape(1, D), eps.reshape(1, 1))


def _sc_aggregate(x_all, src_all, dst_all, ep_all):
    mesh = plsc.VectorSubcoreMesh(core_axis_name="c", subcore_axis_name="s")

    @functools.partial(
        pl.kernel,
        out_type=jax.ShapeDtypeStruct((2, N, D), jnp.float32),
        mesh=mesh,
        scratch_types=[
            pltpu.VMEM((_K,), jnp.int32),
            pltpu.VMEM((_K,), jnp.int32),
            pltpu.VMEM((_K, D), jnp.float32),
            pltpu.VMEM((_K, D), jnp.float32),
            pltpu.VMEM((_ZB, D), jnp.float32),
            pltpu.VMEM_SHARED((N, D), jnp.float32),
            pltpu.SemaphoreType.DMA,
        ],
    )
    def k(x_hbm, src_hbm, dst_hbm, ep_hbm, out_hbm,
          src_v, dst_v, gx_v, ep_v, zero_v, acc_sh, sem):
        cid = lax.axis_index("c")
        sid = lax.axis_index("s")

        zvec = jnp.zeros((16,), jnp.float32)

        def zrow(r, carry):
            for j in range(D // 16):
                zero_v[r, pl.ds(j * 16, 16)] = zvec
            return carry

        lax.fori_loop(0, _ZB, zrow, 0)

        def zacc(t, carry):
            pltpu.sync_copy(zero_v, acc_sh.at[pl.ds(sid * _ZROWS + t * _ZB, _ZB)])
            return carry

        lax.fori_loop(0, _ZROWS // _ZB, zacc, 0)
        plsc.subcore_barrier()

        ebase = cid * E + sid * _EPS_PER

        def chunk(ch, carry):
            off = ebase + ch * _K
            pltpu.sync_copy(src_hbm.at[pl.ds(off, _K)], src_v)
            pltpu.sync_copy(dst_hbm.at[pl.ds(off, _K)], dst_v)
            pltpu.sync_copy(ep_hbm.at[pl.ds(off, _K)], ep_v)
            pltpu.async_copy(x_hbm.at[src_v], gx_v, sem).wait()

            def crow(e, c2):
                for j in range(D // 16):
                    s_ = pl.ds(j * 16, 16)
                    gx_v[e, s_] = jnp.maximum(gx_v[e, s_] + ep_v[e, s_], 0.0)
                return c2

            lax.fori_loop(0, _K, crow, 0)
            pltpu.sync_copy(gx_v, acc_sh.at[dst_v], add=True)
            return carry

        lax.fori_loop(0, _NCH, chunk, 0)
        plsc.subcore_barrier()

        def dump(t, carry):
            r0 = sid * _ZROWS + t * _ZB
            pltpu.sync_copy(acc_sh.at[pl.ds(r0, _ZB)],
                            out_hbm.at[cid, pl.ds(r0, _ZB)])
            return carry

        lax.fori_loop(0, _ZROWS // _ZB, dump, 0)

    return k(x_all, src_all, dst_all, ep_all)


def kernel(x_author, x_paper, edge_index_a2p, edge_index_p2a,
           edge_attr_a2p, edge_attr_p2a, W_edge, b_edge, W_nn, b_nn, eps):
    src_all = jnp.concatenate([
        edge_index_a2p[0].astype(jnp.int32),
        edge_index_p2a[0].astype(jnp.int32) + N,
    ])
    dst_all = jnp.concatenate([
        edge_index_a2p[1].astype(jnp.int32),
        edge_index_p2a[1].astype(jnp.int32),
    ])
    ea_all = jnp.concatenate([edge_attr_a2p, edge_attr_p2a], axis=0)
    ep_all = _eproj(ea_all, W_edge, b_edge)

    xa, xp = x_author, x_paper
    for layer in range(2):
        x_all = jnp.concatenate([xa, xp], axis=0)
        agg = _sc_aggregate(x_all, src_all, dst_all, ep_all)
        x_stack = jnp.concatenate([xp, xa], axis=0)
        new_stack = _out_transform(x_stack, agg.reshape(2 * N, D),
                                   W_nn, b_nn, eps, relu=(layer == 0))
        xp, xa = new_stack[:N], new_stack[N:]
    return (xa, xp)


# speedup vs baseline: 2.1272x; 2.1272x over previous
import functools

import jax
import jax.numpy as jnp
from jax import lax
from jax.experimental import pallas as pl
from jax.experimental.pallas import tpu as pltpu
from jax.experimental.pallas import tpu_sc as plsc

N = 10000
E = 320000
D = 128
DE = 16

_NSUB = 16
_EPS_PER = E // _NSUB
_K = 80
_NCH = _EPS_PER // _K
_PADN = 10240
_ZROWS = _PADN // _NSUB
_ZB = 128


def _eproj(ea_all, w_edge, b_edge):
    bm = 1280
    grid = (ea_all.shape[0] // bm,)

    def body(ea_ref, w_ref, b_ref, o_ref):
        o_ref[...] = (
            jnp.dot(ea_ref[...], w_ref[...], preferred_element_type=jnp.float32)
            + b_ref[...]
        )

    return pl.pallas_call(
        body,
        grid=grid,
        in_specs=[
            pl.BlockSpec((bm, DE), lambda i: (i, 0)),
            pl.BlockSpec((DE, D), lambda i: (0, 0)),
            pl.BlockSpec((1, D), lambda i: (0, 0)),
        ],
        out_specs=pl.BlockSpec((bm, D), lambda i: (i, 0)),
        out_shape=jax.ShapeDtypeStruct((ea_all.shape[0], D), jnp.float32),
    )(ea_all, w_edge, b_edge.reshape(1, D))


def _out_transform(x_stack, agg_stack, w_nn, b_nn, eps, relu):
    bm = 1000
    grid = (x_stack.shape[0] // bm,)

    def body(x_ref, a_ref, w_ref, b_ref, e_ref, o_ref):
        z = (1.0 + e_ref[0, 0]) * x_ref[...] + a_ref[...]
        r = jnp.dot(z, w_ref[...], preferred_element_type=jnp.float32) + b_ref[...]
        o_ref[...] = jnp.maximum(r, 0.0) if relu else r

    return pl.pallas_call(
        body,
        grid=grid,
        in_specs=[
            pl.BlockSpec((bm, D), lambda i: (i, 0)),
            pl.BlockSpec((bm, D), lambda i: (i, 0)),
            pl.BlockSpec((D, D), lambda i: (0, 0)),
            pl.BlockSpec((1, D), lambda i: (0, 0)),
            pl.BlockSpec((1, 1), lambda i: (0, 0)),
        ],
        out_specs=pl.BlockSpec((bm, D), lambda i: (i, 0)),
        out_shape=jax.ShapeDtypeStruct((x_stack.shape[0], D), jnp.float32),
    )(x_stack, agg_stack, w_nn, b_nn.reshape(1, D), eps.reshape(1, 1))


def _sc_aggregate(x_all, src_all, dst_all, ep_all):
    mesh = plsc.VectorSubcoreMesh(core_axis_name="c", subcore_axis_name="s")

    @functools.partial(
        pl.kernel,
        out_type=jax.ShapeDtypeStruct((2, _PADN, D), jnp.float32),
        mesh=mesh,
        scratch_types=[
            pltpu.VMEM((_K,), jnp.int32),
            pltpu.VMEM((_K,), jnp.int32),
            pltpu.VMEM((_K, D), jnp.float32),
            pltpu.VMEM((_K, D), jnp.float32),
            pltpu.VMEM((_ZB, D), jnp.float32),
            pltpu.VMEM_SHARED((_PADN, D), jnp.float32),
            pltpu.SemaphoreType.DMA,
        ],
    )
    def k(x_hbm, src_hbm, dst_hbm, ep_hbm, out_hbm,
          src_v, dst_v, gx_v, ep_v, zero_v, acc_sh, sem):
        cid = lax.axis_index("c")
        sid = lax.axis_index("s")

        zvec = jnp.zeros((16,), jnp.float32)

        def zrow(r, carry):
            for j in range(D // 16):
                zero_v[r, pl.ds(j * 16, 16)] = zvec
            return carry

        lax.fori_loop(0, _ZB, zrow, 0)

        def zacc(t, carry):
            pltpu.sync_copy(zero_v, acc_sh.at[pl.ds(sid * _ZROWS + t * _ZB, _ZB)])
            return carry

        lax.fori_loop(0, _ZROWS // _ZB, zacc, 0)
        plsc.subcore_barrier()

        ebase = cid * E + sid * _EPS_PER

        def chunk(ch, carry):
            off = ebase + ch * _K
            pltpu.sync_copy(src_hbm.at[pl.ds(off, _K)], src_v)
            pltpu.sync_copy(dst_hbm.at[pl.ds(off, _K)], dst_v)
            pltpu.sync_copy(ep_hbm.at[pl.ds(off, _K)], ep_v)
            pltpu.async_copy(x_hbm.at[src_v], gx_v, sem).wait()

            def crow(e, c2):
                for j in range(D // 16):
                    s_ = pl.ds(j * 16, 16)
                    gx_v[e, s_] = jnp.maximum(gx_v[e, s_] + ep_v[e, s_], 0.0)
                return c2

            lax.fori_loop(0, _K, crow, 0)
            pltpu.sync_copy(gx_v, acc_sh.at[dst_v], add=True)
            return carry

        lax.fori_loop(0, _NCH, chunk, 0)
        plsc.subcore_barrier()

        def dump(t, carry):
            r0 = sid * _ZROWS + t * _ZB
            pltpu.sync_copy(acc_sh.at[pl.ds(r0, _ZB)],
                            out_hbm.at[cid, pl.ds(r0, _ZB)])
            return carry

        lax.fori_loop(0, _ZROWS // _ZB, dump, 0)

    return k(x_all, src_all, dst_all, ep_all)


def kernel(x_author, x_paper, edge_index_a2p, edge_index_p2a,
           edge_attr_a2p, edge_attr_p2a, W_edge, b_edge, W_nn, b_nn, eps):
    src_all = jnp.concatenate([
        edge_index_a2p[0].astype(jnp.int32),
        edge_index_p2a[0].astype(jnp.int32) + N,
    ])
    dst_all = jnp.concatenate([
        edge_index_a2p[1].astype(jnp.int32),
        edge_index_p2a[1].astype(jnp.int32),
    ])
    ea_all = jnp.concatenate([edge_attr_a2p, edge_attr_p2a], axis=0)
    ep_all = _eproj(ea_all, W_edge, b_edge)

    xa, xp = x_author, x_paper
    for layer in range(2):
        x_all = jnp.concatenate([xa, xp], axis=0)
        agg = _sc_aggregate(x_all, src_all, dst_all, ep_all)[:, :N, :]
        x_stack = jnp.concatenate([xp, xa], axis=0)
        new_stack = _out_transform(x_stack, agg.reshape(2 * N, D),
                                   W_nn, b_nn, eps, relu=(layer == 0))
        xp, xa = new_stack[:N], new_stack[N:]
    return (xa, xp)


# speedup vs baseline: 3.2589x; 1.5320x over previous
import functools

import jax
import jax.numpy as jnp
from jax import lax
from jax.experimental import pallas as pl
from jax.experimental.pallas import tpu as pltpu
from jax.experimental.pallas import tpu_sc as plsc

N = 10000
E = 320000
D = 128
DE = 16

_NSUB = 16
_EPS_PER = E // _NSUB
_K = 80
_NCH = _EPS_PER // _K
_PADN = 10240
_NCHP = 256
_ZROWS = _PADN // _NSUB
_ZB = 128


def _eproj(ea_all, w_edge, b_edge):
    bm = 1280
    grid = (ea_all.shape[0] // bm,)

    def body(ea_ref, w_ref, b_ref, o_ref):
        o_ref[...] = (
            jnp.dot(ea_ref[...], w_ref[...], preferred_element_type=jnp.float32)
            + b_ref[...]
        )

    return pl.pallas_call(
        body,
        grid=grid,
        in_specs=[
            pl.BlockSpec((bm, DE), lambda i: (i, 0)),
            pl.BlockSpec((DE, D), lambda i: (0, 0)),
            pl.BlockSpec((1, D), lambda i: (0, 0)),
        ],
        out_specs=pl.BlockSpec((bm, D), lambda i: (i, 0)),
        out_shape=jax.ShapeDtypeStruct((ea_all.shape[0], D), jnp.float32),
    )(ea_all, w_edge, b_edge.reshape(1, D))


def _out_transform(x_stack, agg_stack, w_nn, b_nn, eps, relu):
    bm = 1000
    grid = (x_stack.shape[0] // bm,)

    def body(x_ref, a_ref, w_ref, b_ref, e_ref, o_ref):
        z = (1.0 + e_ref[0, 0]) * x_ref[...] + a_ref[...]
        r = jnp.dot(z, w_ref[...], preferred_element_type=jnp.float32) + b_ref[...]
        o_ref[...] = jnp.maximum(r, 0.0) if relu else r

    return pl.pallas_call(
        body,
        grid=grid,
        in_specs=[
            pl.BlockSpec((bm, D), lambda i: (i, 0)),
            pl.BlockSpec((bm, D), lambda i: (i, 0)),
            pl.BlockSpec((D, D), lambda i: (0, 0)),
            pl.BlockSpec((1, D), lambda i: (0, 0)),
            pl.BlockSpec((1, 1), lambda i: (0, 0)),
        ],
        out_specs=pl.BlockSpec((bm, D), lambda i: (i, 0)),
        out_shape=jax.ShapeDtypeStruct((x_stack.shape[0], D), jnp.float32),
    )(x_stack, agg_stack, w_nn, b_nn.reshape(1, D), eps.reshape(1, 1))


def _sc_aggregate(x_all, src_all, dst_all, ep_all):
    mesh = plsc.VectorSubcoreMesh(core_axis_name="c", subcore_axis_name="s")

    @functools.partial(
        pl.kernel,
        out_type=jax.ShapeDtypeStruct((2, _PADN, D), jnp.float32),
        mesh=mesh,
        scratch_types=[
            pltpu.VMEM((2, 2, _K), jnp.int32),
            pltpu.VMEM((2, _K, D), jnp.float32),
            pltpu.VMEM((2, _K, D), jnp.float32),
            pltpu.VMEM_SHARED((_PADN, D), jnp.float32),
            pltpu.SemaphoreType.DMA,
            pltpu.SemaphoreType.DMA,
            pltpu.SemaphoreType.DMA,
            pltpu.SemaphoreType.DMA,
            pltpu.SemaphoreType.DMA,
            pltpu.SemaphoreType.DMA,
            pltpu.SemaphoreType.DMA,
        ],
    )
    def k(x_hbm, src_hbm, dst_hbm, ep_hbm, out_hbm,
          sd_v, gx_v, ep_v, acc_sh,
          sem_i, sem_g0, sem_g1, sem_e0, sem_e1, sem_s0, sem_s1):
        cid = lax.axis_index("c")
        sid = lax.axis_index("s")
        sem_g = (sem_g0, sem_g1)
        sem_e = (sem_e0, sem_e1)
        sem_s = (sem_s0, sem_s1)

        ebase = (cid * _NSUB + sid) * _EPS_PER

        zvec = jnp.zeros((16,), jnp.float32)

        def zrow(r, carry):
            for j in range(D // 16):
                gx_v[1, r, pl.ds(j * 16, 16)] = zvec
            return carry

        lax.fori_loop(0, _K, zrow, 0)

        def zacc(t, carry):
            pltpu.sync_copy(gx_v.at[1],
                            acc_sh.at[pl.ds(sid * _ZROWS + t * _K, _K)])
            return carry

        lax.fori_loop(0, _ZROWS // _K, zacc, 0)

        def issue_idx(ch, sl):
            off = ebase + ch * _K
            pltpu.async_copy(src_hbm.at[pl.ds(off, _K)], sd_v.at[sl, 0], sem_i)
            pltpu.async_copy(dst_hbm.at[pl.ds(off, _K)], sd_v.at[sl, 1], sem_i)

        def wait_idx(ch, sl):
            off = ebase + ch * _K
            pltpu.make_async_copy(src_hbm.at[pl.ds(off, _K)], sd_v.at[sl, 0],
                                  sem_i).wait()
            pltpu.make_async_copy(dst_hbm.at[pl.ds(off, _K)], sd_v.at[sl, 1],
                                  sem_i).wait()

        def issue_in(ch, sl):
            pltpu.async_copy(x_hbm.at[sd_v.at[sl, 0]], gx_v.at[sl], sem_g[sl])
            pltpu.async_copy(ep_hbm.at[pl.ds(ebase + ch * _K, _K)],
                             ep_v.at[sl], sem_e[sl])

        def wait_in(ch, sl):
            pltpu.make_async_copy(x_hbm.at[sd_v.at[sl, 0]], gx_v.at[sl],
                                  sem_g[sl]).wait()
            pltpu.make_async_copy(ep_hbm.at[pl.ds(ebase + ch * _K, _K)],
                                  ep_v.at[sl], sem_e[sl]).wait()

        def compute(sl):
            def crow(e, c2):
                for j in range(D // 16):
                    s_ = pl.ds(j * 16, 16)
                    gx_v[sl, e, s_] = jnp.maximum(
                        gx_v[sl, e, s_] + ep_v[sl, e, s_], 0.0)
                return c2

            lax.fori_loop(0, _K, crow, 0)

        def scatter(sl):
            pltpu.async_copy(gx_v.at[sl], acc_sh.at[sd_v.at[sl, 1]],
                             sem_s[sl], add=True)

        def wait_scatter(sl):
            pltpu.make_async_copy(gx_v.at[sl], acc_sh.at[sd_v.at[sl, 1]],
                                  sem_s[sl]).wait()

        issue_idx(0, 0)
        wait_idx(0, 0)
        issue_in(0, 0)
        pltpu.async_copy(gx_v.at[1], acc_sh.at[sd_v.at[0, 1]], sem_s1,
                         add=True)
        plsc.subcore_barrier()

        def phase(ph, sl, last):
            nsl = 1 - sl
            wait_scatter(nsl)

            @pl.when(jnp.logical_not(last))
            def _():
                issue_idx(ph + 1, nsl)

            wait_in(ph, sl)
            compute(sl)

            @pl.when(jnp.logical_not(last))
            def _():
                wait_idx(ph + 1, nsl)
                issue_in(ph + 1, nsl)

            scatter(sl)

        def step(t, carry):
            phase(2 * t, 0, jnp.bool_(False))
            phase(2 * t + 1, 1, t >= _NCH // 2 - 1)
            return carry

        lax.fori_loop(0, _NCH // 2, step, 0)
        wait_scatter(1)
        plsc.subcore_barrier()

        def dump(t, carry):
            r0 = sid * _ZROWS + t * _ZB
            pltpu.sync_copy(acc_sh.at[pl.ds(r0, _ZB)],
                            out_hbm.at[cid, pl.ds(r0, _ZB)])
            return carry

        lax.fori_loop(0, _ZROWS // _ZB, dump, 0)

    return k(x_all, src_all, dst_all, ep_all)


def kernel(x_author, x_paper, edge_index_a2p, edge_index_p2a,
           edge_attr_a2p, edge_attr_p2a, W_edge, b_edge, W_nn, b_nn, eps):
    src_all = jnp.concatenate([
        edge_index_a2p[0].astype(jnp.int32),
        edge_index_p2a[0].astype(jnp.int32) + N,
    ])
    dst_all = jnp.concatenate([
        edge_index_a2p[1].astype(jnp.int32),
        edge_index_p2a[1].astype(jnp.int32),
    ])
    ea_all = jnp.concatenate([edge_attr_a2p, edge_attr_p2a], axis=0)
    ep_all = _eproj(ea_all, W_edge, b_edge)

    xa, xp = x_author, x_paper
    for layer in range(2):
        x_all = jnp.concatenate([xa, xp], axis=0)
        agg = _sc_aggregate(x_all, src_all, dst_all, ep_all)[:, :N, :]
        x_stack = jnp.concatenate([xp, xa], axis=0)
        new_stack = _out_transform(x_stack, agg.reshape(2 * N, D),
                                   W_nn, b_nn, eps, relu=(layer == 0))
        xp, xa = new_stack[:N], new_stack[N:]
    return (xa, xp)
